# baseline (speedup 1.0000x reference)
import jax
import jax.numpy as jnp
from jax import lax
from jax.experimental import pallas as pl
from jax.experimental.pallas import tpu as pltpu
from jax.experimental.pallas import tpu_sc as plsc

EMB = 256
N = 160000
NCORES = 2
NSUB = 16
NW = NCORES * NSUB
PER = 5120
NPAD = NW * PER
C = 128
D0, D1, D2 = 5, 6, 2
NCOMB = D0 * D1 * D2
LANES = 16


def _sc_body(a_hbm, b_hbm, c_hbm, w0_hbm, w1_hbm, w2_hbm, out_hbm,
             av, bv, cv, idxc, w0v, w1v, w2v, combv, combs, outv,
             gsem, wsem):
    cid = lax.axis_index("c")
    sid = lax.axis_index("s")
    wid = cid * NSUB + sid
    base = wid * PER
    cnt = jnp.minimum(PER, N - base)
    nchunks = cnt // C

    pltpu.async_copy(w0_hbm, w0v, gsem.at[0])
    pltpu.async_copy(w1_hbm, w1v, gsem.at[0])
    pltpu.async_copy(w2_hbm, w2v, gsem.at[0])
    pltpu.async_copy(a_hbm.at[pl.ds(base, PER)], av, gsem.at[1])
    pltpu.async_copy(b_hbm.at[pl.ds(base, PER)], bv, gsem.at[1])
    pltpu.async_copy(c_hbm.at[pl.ds(base, PER)], cv, gsem.at[1])
    pltpu.make_async_copy(w0_hbm, w0v, gsem.at[0]).wait()
    pltpu.make_async_copy(w1_hbm, w1v, gsem.at[0]).wait()
    pltpu.make_async_copy(w2_hbm, w2v, gsem.at[0]).wait()

    iot = lax.iota(jnp.int32, LANES)

    @pl.when(sid < NCOMB // 4)
    def _build_part():
        start = sid * 4

        def build(t, carry):
            k = start + t
            a = k // (D1 * D2)
            r = k - a * (D1 * D2)
            b = r // D2
            c = r - b * D2
            af = jnp.full((LANES,), a, jnp.int32)
            bf = jnp.full((LANES,), b, jnp.int32)
            cf = jnp.full((LANES,), c, jnp.int32)
            kf = jnp.full((LANES,), k, jnp.int32)
            for j in range(EMB // LANES):
                col = iot + j * LANES
                v = (plsc.load_gather(w0v, [af, col]) +
                     plsc.load_gather(w1v, [bf, col]) +
                     plsc.load_gather(w2v, [cf, col]))
                plsc.store_scatter(combv, [kf, col], v)
            return carry

        lax.fori_loop(0, 4, build, 0)
        pltpu.sync_copy(combv.at[pl.ds(start, 4)], combs.at[pl.ds(start, 4)])

    plsc.subcore_barrier()

    pltpu.make_async_copy(a_hbm.at[pl.ds(base, PER)], av, gsem.at[1]).wait()
    pltpu.make_async_copy(b_hbm.at[pl.ds(base, PER)], bv, gsem.at[1]).wait()
    pltpu.make_async_copy(c_hbm.at[pl.ds(base, PER)], cv, gsem.at[1]).wait()

    def idx_chunk(i, b):
        lb = i * C

        def grp(g, carry):
            off = lb + g * LANES
            a16 = av[pl.ds(off, LANES)]
            b16 = bv[pl.ds(off, LANES)]
            c16 = cv[pl.ds(off, LANES)]
            idxc[b, pl.ds(g * LANES, LANES)] = (
                a16 * (D1 * D2) + b16 * D2 + c16)
            return carry

        lax.fori_loop(0, C // LANES, grp, 0)

    def issue_gather(i, b):
        pltpu.async_copy(combs.at[idxc.at[b]], outv.at[b], gsem.at[b])

    def wait_gather(i, b):
        pltpu.make_async_copy(combs.at[idxc.at[b]], outv.at[b],
                              gsem.at[b]).wait()

    def issue_write(i, b):
        pltpu.async_copy(outv.at[b], out_hbm.at[pl.ds(base + i * C, C)],
                         wsem.at[b])

    def wait_write(i, b):
        pltpu.make_async_copy(outv.at[b], out_hbm.at[pl.ds(base + i * C, C)],
                              wsem.at[b]).wait()

    idx_chunk(0, 0)
    issue_gather(0, 0)

    def pipe(ip, carry):
        for b in range(2):
            i = ip * 2 + b
            b2 = 1 - b

            @pl.when(i >= 1)
            def _free_other():
                wait_write(i - 1, b2)

            @pl.when(i < nchunks - 1)
            def _prefetch_next():
                idx_chunk(i + 1, b2)
                issue_gather(i + 1, b2)

            wait_gather(i, b)
            issue_write(i, b)
        return carry

    lax.fori_loop(0, nchunks // 2, pipe, 0)
    wait_write(nchunks - 1, 1)


@jax.jit
def _run(a, b, c, w0, w1, w2):
    mesh = plsc.VectorSubcoreMesh(core_axis_name="c", subcore_axis_name="s")
    f = pl.kernel(
        _sc_body,
        out_type=jax.ShapeDtypeStruct((N, EMB), jnp.float32),
        mesh=mesh,
        scratch_types=[
            pltpu.VMEM((PER,), jnp.int32),
            pltpu.VMEM((PER,), jnp.int32),
            pltpu.VMEM((PER,), jnp.int32),
            pltpu.VMEM((2, C), jnp.int32),
            pltpu.VMEM((D0, EMB), jnp.float32),
            pltpu.VMEM((D1, EMB), jnp.float32),
            pltpu.VMEM((D2, EMB), jnp.float32),
            pltpu.VMEM((NCOMB, EMB), jnp.float32),
            pltpu.VMEM_SHARED((NCOMB, EMB), jnp.float32),
            pltpu.VMEM((2, C, EMB), jnp.float32),
            pltpu.SemaphoreType.DMA((2,)),
            pltpu.SemaphoreType.DMA((2,)),
        ],
        compiler_params=pltpu.CompilerParams(
            use_tc_tiling_on_sc=False, needs_layout_passes=False),
    )
    return f(a, b, c, w0, w1, w2)


def kernel(edge_attr, w0, w1, w2):
    pad = NPAD - N
    a = jnp.pad(edge_attr[:, 0], (0, pad))
    b = jnp.pad(edge_attr[:, 1], (0, pad))
    c = jnp.pad(edge_attr[:, 2], (0, pad))
    return _run(a, b, c, w0, w1, w2)
